# baseline (speedup 1.0000x reference)
import functools

import jax
import jax.numpy as jnp
from jax import lax
from jax.experimental import pallas as pl
from jax.experimental.pallas import tpu as pltpu
from jax.experimental.pallas import tpu_sc as plsc

N_ANC = 20000
N_GT = 128
NW = 32
LANES = 16
K = 5
CHUNK = 640
NBLK = CHUNK // (K * LANES)

_LN2 = 0.6931471805599453
_SQRT2 = 1.4142135623730951


def _vlog(x):
    bits = plsc.bitcast(x, jnp.int32)
    e = jnp.right_shift(bits, 23) - 127
    m = plsc.bitcast((bits & 0x007FFFFF) | 0x3F800000, jnp.float32)
    big = m > _SQRT2
    m = jnp.where(big, m * 0.5, m)
    e = jnp.where(big, e + 1, e)
    t = (m - 1.0) / (m + 1.0)
    t2 = t * t
    p = t * (2.0 + t2 * (2.0 / 3.0 + t2 * (0.4 + t2 * (2.0 / 7.0 + t2 * (2.0 / 9.0)))))
    return e.astype(jnp.float32) * _LN2 + p


def _sc_body(anc_h, gt_h, lbl_h, dx_h, dy_h, dw_h, dh_h,
             anc_v, gt_v, lbl_v, dx_v, dy_v, dw_v, dh_v, sem):
    wid = lax.axis_index("s") * 2 + lax.axis_index("c")
    base = jnp.minimum(wid * CHUNK, N_ANC - CHUNK)

    in0 = pltpu.make_async_copy(anc_h.at[pl.ds(base, CHUNK), :], anc_v, sem)
    in1 = pltpu.make_async_copy(gt_h, gt_v, sem)
    in0.start()
    in1.start()
    in0.wait()
    in1.wait()

    iota = jnp.arange(LANES, dtype=jnp.int32)
    c0 = jnp.zeros((LANES,), jnp.int32)
    c1 = c0 + 1
    c2 = c0 + 2
    c3 = c0 + 3
    c4 = c0 + 4

    def blk_body(b, _):
        off = b * (K * LANES)
        ax1, ay1, ax2, ay2, area = [], [], [], [], []
        for k in range(K):
            lidx = off + k * LANES + iota
            ax1.append(plsc.load_gather(anc_v, [lidx, c0]))
            ay1.append(plsc.load_gather(anc_v, [lidx, c1]))
            ax2.append(plsc.load_gather(anc_v, [lidx, c2]))
            ay2.append(plsc.load_gather(anc_v, [lidx, c3]))
            area.append(((ax2[k] - ax1[k]) + 1.0) * ((ay2[k] - ay1[k]) + 1.0))

        def j_body(j, carry):
            bo = list(carry[:K])
            bi = list(carry[K:])
            jv = jnp.broadcast_to(j, (LANES,)).astype(jnp.int32)
            gx1 = plsc.load_gather(gt_v, [jv, c0])
            gy1 = plsc.load_gather(gt_v, [jv, c1])
            gx2 = plsc.load_gather(gt_v, [jv, c2])
            gy2 = plsc.load_gather(gt_v, [jv, c3])
            gag = ((gx2 - gx1) + 1.0) * ((gy2 - gy1) + 1.0)
            for k in range(K):
                iw = (jnp.minimum(ax2[k], gx2) - jnp.maximum(ax1[k], gx1)) + 1.0
                ih = (jnp.minimum(ay2[k], gy2) - jnp.maximum(ay1[k], gy1)) + 1.0
                inter = jnp.maximum(iw, 0.0) * jnp.maximum(ih, 0.0)
                union = (area[k] + gag) - inter
                ov = inter / union
                upd = ov > bo[k]
                bo[k] = jnp.where(upd, ov, bo[k])
                bi[k] = jnp.where(upd, jv, bi[k])
            return tuple(bo) + tuple(bi)

        init = tuple(jnp.full((LANES,), -1.0, jnp.float32) for _ in range(K)) \
            + tuple(jnp.zeros((LANES,), jnp.int32) for _ in range(K))
        res = lax.fori_loop(0, N_GT, j_body, init)
        bo = res[:K]
        bi = res[K:]

        for k in range(K):
            gx1 = plsc.load_gather(gt_v, [bi[k], c0])
            gy1 = plsc.load_gather(gt_v, [bi[k], c1])
            gx2 = plsc.load_gather(gt_v, [bi[k], c2])
            gy2 = plsc.load_gather(gt_v, [bi[k], c3])
            gcls = plsc.load_gather(gt_v, [bi[k], c4])
            gt_w = (gx2 - gx1) + 1.0
            gt_h = (gy2 - gy1) + 1.0
            gt_cx = gx1 + 0.5 * gt_w
            gt_cy = gy1 + 0.5 * gt_h
            ex_w = (ax2[k] - ax1[k]) + 1.0
            ex_h = (ay2[k] - ay1[k]) + 1.0
            ex_cx = ax1[k] + 0.5 * ex_w
            ex_cy = ay1[k] + 0.5 * ex_h
            lbl = jnp.where(bo[k] >= 0.5, gcls,
                            jnp.where(bo[k] < 0.4, 0.0, -1.0))
            sl = pl.ds(off + k * LANES, LANES)
            lbl_v[sl] = lbl
            dx_v[sl] = (gt_cx - ex_cx) / ex_w
            dy_v[sl] = (gt_cy - ex_cy) / ex_h
            dw_v[sl] = _vlog(gt_w / ex_w)
            dh_v[sl] = _vlog(gt_h / ex_h)
        return 0

    lax.fori_loop(0, NBLK, blk_body, 0)

    outs = [pltpu.make_async_copy(v, h.at[pl.ds(base, CHUNK)], sem)
            for v, h in ((lbl_v, lbl_h), (dx_v, dx_h), (dy_v, dy_h),
                         (dw_v, dw_h), (dh_v, dh_h))]
    for o in outs:
        o.start()
    for o in outs:
        o.wait()


_sc_call = functools.partial(
    pl.kernel,
    out_type=[jax.ShapeDtypeStruct((N_ANC,), jnp.float32)] * 5,
    mesh=plsc.VectorSubcoreMesh(core_axis_name="c", subcore_axis_name="s",
                                num_cores=2, num_subcores=16),
    compiler_params=pltpu.CompilerParams(needs_layout_passes=False),
    scratch_types=(
        [pltpu.VMEM((CHUNK, 4), jnp.float32),
         pltpu.VMEM((N_GT, 5), jnp.float32)]
        + [pltpu.VMEM((CHUNK,), jnp.float32)] * 5
        + [pltpu.SemaphoreType.DMA]
    ),
)(_sc_body)


def kernel(anchors, image_shape, gt_boxes):
    anc = anchors[0].astype(jnp.float32)
    gt = gt_boxes[0].astype(jnp.float32)
    lbl, dx, dy, dw, dh = _sc_call(anc, gt)
    labels = lbl[None]
    bbox = jnp.stack([dx, dy, dw, dh], axis=-1)[None]
    return labels, bbox


# speedup vs baseline: 1.2650x; 1.2650x over previous
import functools

import jax
import jax.numpy as jnp
from jax import lax
from jax.experimental import pallas as pl
from jax.experimental.pallas import tpu as pltpu
from jax.experimental.pallas import tpu_sc as plsc

N_ANC = 20000
N_GT = 128
NW = 32
LANES = 16
K = 4
CHUNK = 640
NBLK = CHUNK // (K * LANES)

_LN2 = 0.6931471805599453
_SQRT2 = 1.4142135623730951


def _vlog(x):
    bits = plsc.bitcast(x, jnp.int32)
    e = jnp.right_shift(bits, 23) - 127
    m = plsc.bitcast((bits & 0x007FFFFF) | 0x3F800000, jnp.float32)
    big = m > _SQRT2
    m = jnp.where(big, m * 0.5, m)
    e = jnp.where(big, e + 1, e)
    t = (m - 1.0) / (m + 1.0)
    t2 = t * t
    p = t * (2.0 + t2 * (2.0 / 3.0 + t2 * (0.4 + t2 * (2.0 / 7.0 + t2 * (2.0 / 9.0)))))
    return e.astype(jnp.float32) * _LN2 + p


def _sc_body(ax1_h, ay1_h, ax2_h, ay2_h, g0_h, g1_h, g2_h, g3_h, g4_h,
             lbl_h, dx_h, dy_h, dw_h, dh_h,
             ax1_v, ay1_v, ax2_v, ay2_v, gv0, gv1, gv2, gv3, gv4,
             lbl_v, dx_v, dy_v, dw_v, dh_v, sem):
    wid = lax.axis_index("s") * 2 + lax.axis_index("c")
    base = jnp.minimum(wid * CHUNK, N_ANC - CHUNK)

    ins = [pltpu.make_async_copy(h.at[pl.ds(base, CHUNK)], v, sem)
           for h, v in ((ax1_h, ax1_v), (ay1_h, ay1_v),
                        (ax2_h, ax2_v), (ay2_h, ay2_v))]
    ins += [pltpu.make_async_copy(h, v, sem)
            for h, v in ((g0_h, gv0), (g1_h, gv1), (g2_h, gv2),
                         (g3_h, gv3), (g4_h, gv4))]
    for c in ins:
        c.start()
    for c in ins:
        c.wait()

    def blk_body(b, _):
        off = b * (K * LANES)
        ax1 = [ax1_v[pl.ds(off + k * LANES, LANES)] for k in range(K)]
        ay1 = [ay1_v[pl.ds(off + k * LANES, LANES)] for k in range(K)]
        ax2 = [ax2_v[pl.ds(off + k * LANES, LANES)] for k in range(K)]
        ay2 = [ay2_v[pl.ds(off + k * LANES, LANES)] for k in range(K)]
        area = [((ax2[k] - ax1[k]) + 1.0) * ((ay2[k] - ay1[k]) + 1.0)
                for k in range(K)]

        def j_body(j, carry):
            bo = list(carry[:K])
            bi = list(carry[K:])
            jv = jnp.broadcast_to(j, (LANES,)).astype(jnp.int32)
            gx1 = plsc.load_gather(gv0, [jv])
            gy1 = plsc.load_gather(gv1, [jv])
            gx2 = plsc.load_gather(gv2, [jv])
            gy2 = plsc.load_gather(gv3, [jv])
            gag = ((gx2 - gx1) + 1.0) * ((gy2 - gy1) + 1.0)
            for k in range(K):
                iw = (jnp.minimum(ax2[k], gx2) - jnp.maximum(ax1[k], gx1)) + 1.0
                ih = (jnp.minimum(ay2[k], gy2) - jnp.maximum(ay1[k], gy1)) + 1.0
                inter = jnp.maximum(iw, 0.0) * jnp.maximum(ih, 0.0)
                union = (area[k] + gag) - inter
                ov = inter / union
                upd = ov > bo[k]
                bo[k] = jnp.where(upd, ov, bo[k])
                bi[k] = jnp.where(upd, jv, bi[k])
            return tuple(bo) + tuple(bi)

        init = tuple(jnp.full((LANES,), -1.0, jnp.float32) for _ in range(K)) \
            + tuple(jnp.zeros((LANES,), jnp.int32) for _ in range(K))
        res = lax.fori_loop(0, N_GT, j_body, init)
        bo = res[:K]
        bi = res[K:]

        for k in range(K):
            gx1 = plsc.load_gather(gv0, [bi[k]])
            gy1 = plsc.load_gather(gv1, [bi[k]])
            gx2 = plsc.load_gather(gv2, [bi[k]])
            gy2 = plsc.load_gather(gv3, [bi[k]])
            gcls = plsc.load_gather(gv4, [bi[k]])
            gt_w = (gx2 - gx1) + 1.0
            gt_h = (gy2 - gy1) + 1.0
            gt_cx = gx1 + 0.5 * gt_w
            gt_cy = gy1 + 0.5 * gt_h
            ex_w = (ax2[k] - ax1[k]) + 1.0
            ex_h = (ay2[k] - ay1[k]) + 1.0
            ex_cx = ax1[k] + 0.5 * ex_w
            ex_cy = ay1[k] + 0.5 * ex_h
            lbl = jnp.where(bo[k] >= 0.5, gcls,
                            jnp.where(bo[k] < 0.4, 0.0, -1.0))
            sl = pl.ds(off + k * LANES, LANES)
            lbl_v[sl] = lbl
            dx_v[sl] = (gt_cx - ex_cx) / ex_w
            dy_v[sl] = (gt_cy - ex_cy) / ex_h
            dw_v[sl] = _vlog(gt_w / ex_w)
            dh_v[sl] = _vlog(gt_h / ex_h)
        return 0

    lax.fori_loop(0, NBLK, blk_body, 0)

    outs = [pltpu.make_async_copy(v, h.at[pl.ds(base, CHUNK)], sem)
            for v, h in ((lbl_v, lbl_h), (dx_v, dx_h), (dy_v, dy_h),
                         (dw_v, dw_h), (dh_v, dh_h))]
    for o in outs:
        o.start()
    for o in outs:
        o.wait()


_sc_call = functools.partial(
    pl.kernel,
    out_type=[jax.ShapeDtypeStruct((N_ANC,), jnp.float32)] * 5,
    mesh=plsc.VectorSubcoreMesh(core_axis_name="c", subcore_axis_name="s",
                                num_cores=2, num_subcores=16),
    compiler_params=pltpu.CompilerParams(needs_layout_passes=False),
    scratch_types=(
        [pltpu.VMEM((CHUNK,), jnp.float32)] * 4
        + [pltpu.VMEM((N_GT,), jnp.float32)] * 5
        + [pltpu.VMEM((CHUNK,), jnp.float32)] * 5
        + [pltpu.SemaphoreType.DMA]
    ),
)(_sc_body)


def kernel(anchors, image_shape, gt_boxes):
    anc = anchors[0].astype(jnp.float32)
    gt = gt_boxes[0].astype(jnp.float32)
    lbl, dx, dy, dw, dh = _sc_call(
        anc[:, 0], anc[:, 1], anc[:, 2], anc[:, 3],
        gt[:, 0], gt[:, 1], gt[:, 2], gt[:, 3], gt[:, 4])
    labels = lbl[None]
    bbox = jnp.stack([dx, dy, dw, dh], axis=-1)[None]
    return labels, bbox


# speedup vs baseline: 1.2690x; 1.0032x over previous
import functools

import jax
import jax.numpy as jnp
from jax import lax
from jax.experimental import pallas as pl
from jax.experimental.pallas import tpu as pltpu
from jax.experimental.pallas import tpu_sc as plsc

N_ANC = 20000
N_GT = 128
NW = 32
LANES = 16
K = 5
CHUNK = 640
NBLK = CHUNK // (K * LANES)

_LN2 = 0.6931471805599453
_SQRT2 = 1.4142135623730951


def _vlog(x):
    bits = plsc.bitcast(x, jnp.int32)
    e = jnp.right_shift(bits, 23) - 127
    m = plsc.bitcast((bits & 0x007FFFFF) | 0x3F800000, jnp.float32)
    big = m > _SQRT2
    m = jnp.where(big, m * 0.5, m)
    e = jnp.where(big, e + 1, e)
    t = (m - 1.0) / (m + 1.0)
    t2 = t * t
    p = t * (2.0 + t2 * (2.0 / 3.0 + t2 * (0.4 + t2 * (2.0 / 7.0 + t2 * (2.0 / 9.0)))))
    return e.astype(jnp.float32) * _LN2 + p


def _sc_body(ax1_h, ay1_h, ax2_h, ay2_h, g0_h, g1_h, g2_h, g3_h, g4_h,
             lbl_h, dx_h, dy_h, dw_h, dh_h,
             ax1_v, ay1_v, ax2_v, ay2_v, gv0, gv1, gv2, gv3, gv4,
             lbl_v, dx_v, dy_v, dw_v, dh_v, sem):
    wid = lax.axis_index("s") * 2 + lax.axis_index("c")
    base = jnp.minimum(wid * CHUNK, N_ANC - CHUNK)

    ins = [pltpu.make_async_copy(h.at[pl.ds(base, CHUNK)], v, sem)
           for h, v in ((ax1_h, ax1_v), (ay1_h, ay1_v),
                        (ax2_h, ax2_v), (ay2_h, ay2_v))]
    ins += [pltpu.make_async_copy(h, v, sem)
            for h, v in ((g0_h, gv0), (g1_h, gv1), (g2_h, gv2),
                         (g3_h, gv3), (g4_h, gv4))]
    for c in ins:
        c.start()
    for c in ins:
        c.wait()

    def blk_body(b, _):
        off = b * (K * LANES)
        ax1 = [ax1_v[pl.ds(off + k * LANES, LANES)] for k in range(K)]
        ay1 = [ay1_v[pl.ds(off + k * LANES, LANES)] for k in range(K)]
        ax2 = [ax2_v[pl.ds(off + k * LANES, LANES)] for k in range(K)]
        ay2 = [ay2_v[pl.ds(off + k * LANES, LANES)] for k in range(K)]
        area = [((ax2[k] - ax1[k]) + 1.0) * ((ay2[k] - ay1[k]) + 1.0)
                for k in range(K)]

        def j_body(j, carry):
            bo = list(carry[:K])
            bi = list(carry[K:])
            jv = jnp.broadcast_to(j, (LANES,)).astype(jnp.int32)
            gx1 = plsc.load_gather(gv0, [jv])
            gy1 = plsc.load_gather(gv1, [jv])
            gx2 = plsc.load_gather(gv2, [jv])
            gy2 = plsc.load_gather(gv3, [jv])
            gag = ((gx2 - gx1) + 1.0) * ((gy2 - gy1) + 1.0)
            for k in range(K):
                iw = (jnp.minimum(ax2[k], gx2) - jnp.maximum(ax1[k], gx1)) + 1.0
                ih = (jnp.minimum(ay2[k], gy2) - jnp.maximum(ay1[k], gy1)) + 1.0
                inter = jnp.maximum(iw, 0.0) * jnp.maximum(ih, 0.0)
                union = (area[k] + gag) - inter
                ov = inter / union
                upd = ov > bo[k]
                bo[k] = jnp.where(upd, ov, bo[k])
                bi[k] = jnp.where(upd, jv, bi[k])
            return tuple(bo) + tuple(bi)

        init = tuple(jnp.full((LANES,), -1.0, jnp.float32) for _ in range(K)) \
            + tuple(jnp.zeros((LANES,), jnp.int32) for _ in range(K))
        res = lax.fori_loop(0, N_GT, j_body, init)
        bo = res[:K]
        bi = res[K:]

        for k in range(K):
            gx1 = plsc.load_gather(gv0, [bi[k]])
            gy1 = plsc.load_gather(gv1, [bi[k]])
            gx2 = plsc.load_gather(gv2, [bi[k]])
            gy2 = plsc.load_gather(gv3, [bi[k]])
            gcls = plsc.load_gather(gv4, [bi[k]])
            gt_w = (gx2 - gx1) + 1.0
            gt_h = (gy2 - gy1) + 1.0
            gt_cx = gx1 + 0.5 * gt_w
            gt_cy = gy1 + 0.5 * gt_h
            ex_w = (ax2[k] - ax1[k]) + 1.0
            ex_h = (ay2[k] - ay1[k]) + 1.0
            ex_cx = ax1[k] + 0.5 * ex_w
            ex_cy = ay1[k] + 0.5 * ex_h
            lbl = jnp.where(bo[k] >= 0.5, gcls,
                            jnp.where(bo[k] < 0.4, 0.0, -1.0))
            sl = pl.ds(off + k * LANES, LANES)
            lbl_v[sl] = lbl
            dx_v[sl] = (gt_cx - ex_cx) / ex_w
            dy_v[sl] = (gt_cy - ex_cy) / ex_h
            dw_v[sl] = _vlog(gt_w / ex_w)
            dh_v[sl] = _vlog(gt_h / ex_h)
        return 0

    lax.fori_loop(0, NBLK, blk_body, 0)

    outs = [pltpu.make_async_copy(v, h.at[pl.ds(base, CHUNK)], sem)
            for v, h in ((lbl_v, lbl_h), (dx_v, dx_h), (dy_v, dy_h),
                         (dw_v, dw_h), (dh_v, dh_h))]
    for o in outs:
        o.start()
    for o in outs:
        o.wait()


_sc_call = functools.partial(
    pl.kernel,
    out_type=[jax.ShapeDtypeStruct((N_ANC,), jnp.float32)] * 5,
    mesh=plsc.VectorSubcoreMesh(core_axis_name="c", subcore_axis_name="s",
                                num_cores=2, num_subcores=16),
    compiler_params=pltpu.CompilerParams(needs_layout_passes=False),
    scratch_types=(
        [pltpu.VMEM((CHUNK,), jnp.float32)] * 4
        + [pltpu.VMEM((N_GT,), jnp.float32)] * 5
        + [pltpu.VMEM((CHUNK,), jnp.float32)] * 5
        + [pltpu.SemaphoreType.DMA]
    ),
)(_sc_body)


def kernel(anchors, image_shape, gt_boxes):
    anc = anchors[0].astype(jnp.float32)
    gt = gt_boxes[0].astype(jnp.float32)
    lbl, dx, dy, dw, dh = _sc_call(
        anc[:, 0], anc[:, 1], anc[:, 2], anc[:, 3],
        gt[:, 0], gt[:, 1], gt[:, 2], gt[:, 3], gt[:, 4])
    labels = lbl[None]
    bbox = jnp.stack([dx, dy, dw, dh], axis=-1)[None]
    return labels, bbox


# speedup vs baseline: 1.2822x; 1.0104x over previous
import functools

import jax
import jax.numpy as jnp
from jax import lax
from jax.experimental import pallas as pl
from jax.experimental.pallas import tpu as pltpu
from jax.experimental.pallas import tpu_sc as plsc

N_ANC = 20000
N_GT = 128
NW = 32
LANES = 16
K = 5
CHUNK = 640
NBLK = CHUNK // (K * LANES)

_LN2 = 0.6931471805599453
_SQRT2 = 1.4142135623730951


def _vlog(x):
    bits = plsc.bitcast(x, jnp.int32)
    e = jnp.right_shift(bits, 23) - 127
    m = plsc.bitcast((bits & 0x007FFFFF) | 0x3F800000, jnp.float32)
    big = m > _SQRT2
    m = jnp.where(big, m * 0.5, m)
    e = jnp.where(big, e + 1, e)
    t = (m - 1.0) / (m + 1.0)
    t2 = t * t
    p = t * (2.0 + t2 * (2.0 / 3.0 + t2 * (0.4 + t2 * (2.0 / 7.0 + t2 * (2.0 / 9.0)))))
    return e.astype(jnp.float32) * _LN2 + p


def _sc_body(ax1_h, ay1_h, ax2_h, ay2_h, g0_h, g1_h, g2_h, g3_h, g4_h,
             lbl_h, dx_h, dy_h, dw_h, dh_h,
             ax1_v, ay1_v, ax2_v, ay2_v, gv0, gv1, gv2, gv3, gv4,
             lbl_v, dx_v, dy_v, dw_v, dh_v, sem):
    wid = lax.axis_index("s") * 2 + lax.axis_index("c")
    base = jnp.minimum(wid * CHUNK, N_ANC - CHUNK)

    ins = [pltpu.make_async_copy(h.at[pl.ds(base, CHUNK)], v, sem)
           for h, v in ((ax1_h, ax1_v), (ay1_h, ay1_v),
                        (ax2_h, ax2_v), (ay2_h, ay2_v))]
    ins += [pltpu.make_async_copy(h, v, sem)
            for h, v in ((g0_h, gv0), (g1_h, gv1), (g2_h, gv2),
                         (g3_h, gv3), (g4_h, gv4))]
    for c in ins:
        c.start()
    for c in ins:
        c.wait()

    def blk_body(b, _):
        off = b * (K * LANES)
        ax1 = [ax1_v[pl.ds(off + k * LANES, LANES)] for k in range(K)]
        ay1 = [ay1_v[pl.ds(off + k * LANES, LANES)] for k in range(K)]
        ax2 = [ax2_v[pl.ds(off + k * LANES, LANES)] for k in range(K)]
        ay2 = [ay2_v[pl.ds(off + k * LANES, LANES)] for k in range(K)]
        area = [((ax2[k] - ax1[k]) + 1.0) * ((ay2[k] - ay1[k]) + 1.0)
                for k in range(K)]
        ax2p = [a + 1.0 for a in ax2]
        ay2p = [a + 1.0 for a in ay2]

        def grp_body(g, carry):
            bo = list(carry[:K])
            bi = list(carry[K:])
            g16v = jnp.broadcast_to(g * LANES, (LANES,)).astype(jnp.int32)
            for l in range(LANES):
                jv = g16v + l
                gx1 = plsc.load_gather(gv0, [jv])
                gy1 = plsc.load_gather(gv1, [jv])
                gx2 = plsc.load_gather(gv2, [jv])
                gy2 = plsc.load_gather(gv3, [jv])
                gag = ((gx2 - gx1) + 1.0) * ((gy2 - gy1) + 1.0)
                gx2p = gx2 + 1.0
                gy2p = gy2 + 1.0
                for k in range(K):
                    iw = jnp.minimum(ax2p[k], gx2p) - jnp.maximum(ax1[k], gx1)
                    ih = jnp.minimum(ay2p[k], gy2p) - jnp.maximum(ay1[k], gy1)
                    inter = jnp.maximum(iw, 0.0) * jnp.maximum(ih, 0.0)
                    union = (area[k] + gag) - inter
                    ov = inter / union
                    upd = ov > bo[k]
                    bo[k] = jnp.where(upd, ov, bo[k])
                    bi[k] = jnp.where(upd, jv, bi[k])
            return tuple(bo) + tuple(bi)

        init = tuple(jnp.full((LANES,), -1.0, jnp.float32) for _ in range(K)) \
            + tuple(jnp.zeros((LANES,), jnp.int32) for _ in range(K))
        res = lax.fori_loop(0, N_GT // LANES, grp_body, init)
        bo = res[:K]
        bi = res[K:]

        for k in range(K):
            gx1 = plsc.load_gather(gv0, [bi[k]])
            gy1 = plsc.load_gather(gv1, [bi[k]])
            gx2 = plsc.load_gather(gv2, [bi[k]])
            gy2 = plsc.load_gather(gv3, [bi[k]])
            gcls = plsc.load_gather(gv4, [bi[k]])
            gt_w = (gx2 - gx1) + 1.0
            gt_h = (gy2 - gy1) + 1.0
            gt_cx = gx1 + 0.5 * gt_w
            gt_cy = gy1 + 0.5 * gt_h
            ex_w = (ax2[k] - ax1[k]) + 1.0
            ex_h = (ay2[k] - ay1[k]) + 1.0
            ex_cx = ax1[k] + 0.5 * ex_w
            ex_cy = ay1[k] + 0.5 * ex_h
            lbl = jnp.where(bo[k] >= 0.5, gcls,
                            jnp.where(bo[k] < 0.4, 0.0, -1.0))
            sl = pl.ds(off + k * LANES, LANES)
            lbl_v[sl] = lbl
            dx_v[sl] = (gt_cx - ex_cx) / ex_w
            dy_v[sl] = (gt_cy - ex_cy) / ex_h
            dw_v[sl] = _vlog(gt_w / ex_w)
            dh_v[sl] = _vlog(gt_h / ex_h)
        return 0

    lax.fori_loop(0, NBLK, blk_body, 0)

    outs = [pltpu.make_async_copy(v, h.at[pl.ds(base, CHUNK)], sem)
            for v, h in ((lbl_v, lbl_h), (dx_v, dx_h), (dy_v, dy_h),
                         (dw_v, dw_h), (dh_v, dh_h))]
    for o in outs:
        o.start()
    for o in outs:
        o.wait()


_sc_call = functools.partial(
    pl.kernel,
    out_type=[jax.ShapeDtypeStruct((N_ANC,), jnp.float32)] * 5,
    mesh=plsc.VectorSubcoreMesh(core_axis_name="c", subcore_axis_name="s",
                                num_cores=2, num_subcores=16),
    compiler_params=pltpu.CompilerParams(needs_layout_passes=False),
    scratch_types=(
        [pltpu.VMEM((CHUNK,), jnp.float32)] * 4
        + [pltpu.VMEM((N_GT,), jnp.float32)] * 5
        + [pltpu.VMEM((CHUNK,), jnp.float32)] * 5
        + [pltpu.SemaphoreType.DMA]
    ),
)(_sc_body)


def kernel(anchors, image_shape, gt_boxes):
    anc = anchors[0].astype(jnp.float32)
    gt = gt_boxes[0].astype(jnp.float32)
    lbl, dx, dy, dw, dh = _sc_call(
        anc[:, 0], anc[:, 1], anc[:, 2], anc[:, 3],
        gt[:, 0], gt[:, 1], gt[:, 2], gt[:, 3], gt[:, 4])
    labels = lbl[None]
    bbox = jnp.stack([dx, dy, dw, dh], axis=-1)[None]
    return labels, bbox


# speedup vs baseline: 1.3236x; 1.0323x over previous
import functools

import jax
import jax.numpy as jnp
from jax import lax
from jax.experimental import pallas as pl
from jax.experimental.pallas import tpu as pltpu
from jax.experimental.pallas import tpu_sc as plsc

N_ANC = 20000
N_GT = 128
NW = 32
LANES = 16
K = 5
CHUNK = 640
NBLK = CHUNK // (K * LANES)

_LN2 = 0.6931471805599453
_SQRT2 = 1.4142135623730951


def _vlog(x):
    bits = plsc.bitcast(x, jnp.int32)
    e = jnp.right_shift(bits, 23) - 127
    m = plsc.bitcast((bits & 0x007FFFFF) | 0x3F800000, jnp.float32)
    big = m > _SQRT2
    m = jnp.where(big, m * 0.5, m)
    e = jnp.where(big, e + 1, e)
    t = (m - 1.0) / (m + 1.0)
    t2 = t * t
    p = t * (2.0 + t2 * (2.0 / 3.0 + t2 * (0.4 + t2 * (2.0 / 7.0 + t2 * (2.0 / 9.0)))))
    return e.astype(jnp.float32) * _LN2 + p


def _sc_body(ax1_h, ay1_h, ax2_h, ay2_h, g0_h, g1_h, g2_h, g3_h, g4_h,
             lbl_h, dx_h, dy_h, dw_h, dh_h,
             ax1_v, ay1_v, ax2_v, ay2_v, gv0, gv1, gv2, gv3, gv4,
             gag_v, gx2p_v, gy2p_v,
             lbl_v, dx_v, dy_v, dw_v, dh_v, sem):
    wid = lax.axis_index("s") * 2 + lax.axis_index("c")
    base = jnp.minimum(wid * CHUNK, N_ANC - CHUNK)

    ins = [pltpu.make_async_copy(h.at[pl.ds(base, CHUNK)], v, sem)
           for h, v in ((ax1_h, ax1_v), (ay1_h, ay1_v),
                        (ax2_h, ax2_v), (ay2_h, ay2_v))]
    ins += [pltpu.make_async_copy(h, v, sem)
            for h, v in ((g0_h, gv0), (g1_h, gv1), (g2_h, gv2),
                         (g3_h, gv3), (g4_h, gv4))]
    for c in ins:
        c.start()
    for c in ins:
        c.wait()

    def tbl_body(t, _):
        sl = pl.ds(t * LANES, LANES)
        tx1 = gv0[sl]
        ty1 = gv1[sl]
        tx2 = gv2[sl]
        ty2 = gv3[sl]
        gag_v[sl] = ((tx2 - tx1) + 1.0) * ((ty2 - ty1) + 1.0)
        gx2p_v[sl] = tx2 + 1.0
        gy2p_v[sl] = ty2 + 1.0
        return 0

    lax.fori_loop(0, N_GT // LANES, tbl_body, 0)

    def blk_body(b, _):
        off = b * (K * LANES)
        ax1 = [ax1_v[pl.ds(off + k * LANES, LANES)] for k in range(K)]
        ay1 = [ay1_v[pl.ds(off + k * LANES, LANES)] for k in range(K)]
        ax2 = [ax2_v[pl.ds(off + k * LANES, LANES)] for k in range(K)]
        ay2 = [ay2_v[pl.ds(off + k * LANES, LANES)] for k in range(K)]
        area = [((ax2[k] - ax1[k]) + 1.0) * ((ay2[k] - ay1[k]) + 1.0)
                for k in range(K)]
        ax2p = [a + 1.0 for a in ax2]
        ay2p = [a + 1.0 for a in ay2]

        def grp_body(g, carry):
            bo = list(carry[:K])
            bi = list(carry[K:])
            g16v = jnp.broadcast_to(g * LANES, (LANES,)).astype(jnp.int32)
            for l in range(LANES):
                jv = g16v + l
                gx1 = plsc.load_gather(gv0, [jv])
                gy1 = plsc.load_gather(gv1, [jv])
                gx2p = plsc.load_gather(gx2p_v, [jv])
                gy2p = plsc.load_gather(gy2p_v, [jv])
                gag = plsc.load_gather(gag_v, [jv])
                for k in range(K):
                    iw = jnp.minimum(ax2p[k], gx2p) - jnp.maximum(ax1[k], gx1)
                    ih = jnp.minimum(ay2p[k], gy2p) - jnp.maximum(ay1[k], gy1)
                    inter = jnp.maximum(iw, 0.0) * jnp.maximum(ih, 0.0)
                    union = (area[k] + gag) - inter
                    ov = inter / union
                    upd = ov > bo[k]
                    bo[k] = jnp.where(upd, ov, bo[k])
                    bi[k] = jnp.where(upd, jv, bi[k])
            return tuple(bo) + tuple(bi)

        init = tuple(jnp.full((LANES,), -1.0, jnp.float32) for _ in range(K)) \
            + tuple(jnp.zeros((LANES,), jnp.int32) for _ in range(K))
        res = lax.fori_loop(0, N_GT // LANES, grp_body, init)
        bo = res[:K]
        bi = res[K:]

        for k in range(K):
            gx1 = plsc.load_gather(gv0, [bi[k]])
            gy1 = plsc.load_gather(gv1, [bi[k]])
            gx2 = plsc.load_gather(gv2, [bi[k]])
            gy2 = plsc.load_gather(gv3, [bi[k]])
            gcls = plsc.load_gather(gv4, [bi[k]])
            gt_w = (gx2 - gx1) + 1.0
            gt_h = (gy2 - gy1) + 1.0
            gt_cx = gx1 + 0.5 * gt_w
            gt_cy = gy1 + 0.5 * gt_h
            ex_w = (ax2[k] - ax1[k]) + 1.0
            ex_h = (ay2[k] - ay1[k]) + 1.0
            ex_cx = ax1[k] + 0.5 * ex_w
            ex_cy = ay1[k] + 0.5 * ex_h
            lbl = jnp.where(bo[k] >= 0.5, gcls,
                            jnp.where(bo[k] < 0.4, 0.0, -1.0))
            sl = pl.ds(off + k * LANES, LANES)
            lbl_v[sl] = lbl
            dx_v[sl] = (gt_cx - ex_cx) / ex_w
            dy_v[sl] = (gt_cy - ex_cy) / ex_h
            dw_v[sl] = _vlog(gt_w / ex_w)
            dh_v[sl] = _vlog(gt_h / ex_h)
        return 0

    lax.fori_loop(0, NBLK, blk_body, 0)

    outs = [pltpu.make_async_copy(v, h.at[pl.ds(base, CHUNK)], sem)
            for v, h in ((lbl_v, lbl_h), (dx_v, dx_h), (dy_v, dy_h),
                         (dw_v, dw_h), (dh_v, dh_h))]
    for o in outs:
        o.start()
    for o in outs:
        o.wait()


_sc_call = functools.partial(
    pl.kernel,
    out_type=[jax.ShapeDtypeStruct((N_ANC,), jnp.float32)] * 5,
    mesh=plsc.VectorSubcoreMesh(core_axis_name="c", subcore_axis_name="s",
                                num_cores=2, num_subcores=16),
    compiler_params=pltpu.CompilerParams(needs_layout_passes=False),
    scratch_types=(
        [pltpu.VMEM((CHUNK,), jnp.float32)] * 4
        + [pltpu.VMEM((N_GT,), jnp.float32)] * 5
        + [pltpu.VMEM((N_GT,), jnp.float32)] * 3
        + [pltpu.VMEM((CHUNK,), jnp.float32)] * 5
        + [pltpu.SemaphoreType.DMA]
    ),
)(_sc_body)


def kernel(anchors, image_shape, gt_boxes):
    anc = anchors[0].astype(jnp.float32)
    gt = gt_boxes[0].astype(jnp.float32)
    lbl, dx, dy, dw, dh = _sc_call(
        anc[:, 0], anc[:, 1], anc[:, 2], anc[:, 3],
        gt[:, 0], gt[:, 1], gt[:, 2], gt[:, 3], gt[:, 4])
    labels = lbl[None]
    bbox = jnp.stack([dx, dy, dw, dh], axis=-1)[None]
    return labels, bbox
